# Initial kernel scaffold; baseline (speedup 1.0000x reference)
import jax
import jax.numpy as jnp
from jax.experimental import pallas as pl


def kernel(atom_in_fea, nbr_fea, nbr_fea_idx, W, b, g1, b1, g2, b2):
    raise NotImplementedError("write your pallas kernel here")


# speedup vs baseline: 1.8704x; 1.8704x over previous
import functools
import jax
import jax.numpy as jnp
from jax import lax
from jax.experimental import pallas as pl
from jax.experimental.pallas import tpu as pltpu
from jax.experimental.pallas import tpu_sc as plsc

AFL = 128
NBR = 16
DF = 2 * AFL + NBR
EPS = 1e-5


def _sc_gather(table, idx_flat):
    E = idx_flat.shape[0]
    NW = 32
    per_w = E // NW
    CH = 80
    n_ch = per_w // CH
    mesh = plsc.VectorSubcoreMesh(core_axis_name="c", subcore_axis_name="s")

    @functools.partial(
        pl.kernel,
        mesh=mesh,
        out_type=jax.ShapeDtypeStruct((E, AFL), jnp.float32),
        scratch_types=[
            pltpu.VMEM((per_w,), jnp.int32),
            pltpu.VMEM((CH, AFL), jnp.float32),
            pltpu.SemaphoreType.DMA,
        ],
    )
    def gather_kernel(table_hbm, idx_hbm, out_hbm, idx_v, rows_v, sem):
        wid = lax.axis_index("s") * 2 + lax.axis_index("c")
        base = pl.multiple_of(wid * per_w, 8)
        pltpu.sync_copy(idx_hbm.at[pl.ds(base, per_w)], idx_v)

        def body(i, carry):
            off = pl.multiple_of(i * CH, 8)
            pltpu.async_copy(
                table_hbm.at[idx_v.at[pl.ds(off, CH)]], rows_v, sem
            ).wait()
            pltpu.sync_copy(rows_v, out_hbm.at[pl.ds(base + off, CH)])
            return carry

        lax.fori_loop(0, n_ch, body, 0)

    return gather_kernel(table, idx_flat)


def _mm_stats_body(atom_ref, ag_ref, nbr_ref, w0_ref, w1_ref, w2_ref, b_ref,
                   g_ref, s_ref, q_ref, *, bn, m):
    sp = jnp.dot(atom_ref[:], w0_ref[:], preferred_element_type=jnp.float32)
    sp = jnp.broadcast_to(sp[:, None, :], (bn, m, DF)).reshape(bn * m, DF)
    g = sp + jnp.dot(ag_ref[:], w1_ref[:], preferred_element_type=jnp.float32)
    g = g + jnp.dot(nbr_ref[:], w2_ref[:], preferred_element_type=jnp.float32)
    g = g + b_ref[:]
    g_ref[:] = g

    @pl.when(pl.program_id(0) == 0)
    def _():
        s_ref[:] = jnp.zeros_like(s_ref)
        q_ref[:] = jnp.zeros_like(q_ref)

    s_ref[:] += jnp.sum(g, axis=0, keepdims=True)
    q_ref[:] += jnp.sum(g * g, axis=0, keepdims=True)


def _apply_body(g_ref, nbr_ref, s_ref, q_ref, g1_ref, b1_ref,
                ns_ref, nn_ref, s2_ref, q2_ref, *, bn, m, nm_total):
    mean = s_ref[:] / nm_total
    var = q_ref[:] / nm_total - mean * mean
    a1 = g1_ref[:] * lax.rsqrt(var + EPS)
    c1 = b1_ref[:] - mean * a1
    a1r = a1.reshape(1, 1, DF)
    c1r = c1.reshape(1, 1, DF)

    gb = g_ref[:] * a1r + c1r

    mx = gb[:, 0, :AFL]
    for j in range(1, m):
        mx = jnp.maximum(mx, gb[:, j, :AFL])
    z = jnp.zeros((bn, AFL), jnp.float32)
    acc = jnp.zeros((bn, AFL), jnp.float32)
    for j in range(m):
        e = jnp.exp(gb[:, j, :AFL] - mx)
        z = z + e
        acc = acc + e * jnp.maximum(gb[:, j, AFL:2 * AFL], 0.0)
    ns = acc / z
    ns_ref[:] = ns
    nn_ref[:] = gb[:, :, 2 * AFL:] + nbr_ref[:]

    @pl.when(pl.program_id(0) == 0)
    def _():
        s2_ref[:] = jnp.zeros_like(s2_ref)
        q2_ref[:] = jnp.zeros_like(q2_ref)

    s2_ref[:] += jnp.sum(ns, axis=0, keepdims=True)
    q2_ref[:] += jnp.sum(ns * ns, axis=0, keepdims=True)


def _bn2_body(atom_ref, ns_ref, s2_ref, q2_ref, g2_ref, b2_ref, out_ref, *, n_total):
    mean = s2_ref[:] / n_total
    var = q2_ref[:] / n_total - mean * mean
    a2 = g2_ref[:] * lax.rsqrt(var + EPS)
    c2 = b2_ref[:] - mean * a2
    out_ref[:] = atom_ref[:] + ns_ref[:] * a2 + c2


def kernel(atom_in_fea, nbr_fea, nbr_fea_idx, W, b, g1, b1, g2, b2):
    N, M = nbr_fea_idx.shape
    E = N * M

    idx_flat = nbr_fea_idx.astype(jnp.int32).reshape(E)
    ag = _sc_gather(atom_in_fea, idx_flat)

    nbr_flat = nbr_fea.reshape(E, NBR)
    w0 = W[:AFL, :]
    w1 = W[AFL:2 * AFL, :]
    w2 = W[2 * AFL:, :]
    b2d = b.reshape(1, DF)

    BN = 200
    R = BN * M
    grid_a = N // BN

    g, s, q = pl.pallas_call(
        functools.partial(_mm_stats_body, bn=BN, m=M),
        grid=(grid_a,),
        in_specs=[
            pl.BlockSpec((BN, AFL), lambda i: (i, 0)),
            pl.BlockSpec((R, AFL), lambda i: (i, 0)),
            pl.BlockSpec((R, NBR), lambda i: (i, 0)),
            pl.BlockSpec((AFL, DF), lambda i: (0, 0)),
            pl.BlockSpec((AFL, DF), lambda i: (0, 0)),
            pl.BlockSpec((NBR, DF), lambda i: (0, 0)),
            pl.BlockSpec((1, DF), lambda i: (0, 0)),
        ],
        out_specs=[
            pl.BlockSpec((R, DF), lambda i: (i, 0)),
            pl.BlockSpec((1, DF), lambda i: (0, 0)),
            pl.BlockSpec((1, DF), lambda i: (0, 0)),
        ],
        out_shape=[
            jax.ShapeDtypeStruct((E, DF), jnp.float32),
            jax.ShapeDtypeStruct((1, DF), jnp.float32),
            jax.ShapeDtypeStruct((1, DF), jnp.float32),
        ],
    )(atom_in_fea, ag, nbr_flat, w0, w1, w2, b2d)

    g3 = g.reshape(N, M, DF)
    ns, nn, s2, q2 = pl.pallas_call(
        functools.partial(_apply_body, bn=BN, m=M, nm_total=float(E)),
        grid=(grid_a,),
        in_specs=[
            pl.BlockSpec((BN, M, DF), lambda i: (i, 0, 0)),
            pl.BlockSpec((BN, M, NBR), lambda i: (i, 0, 0)),
            pl.BlockSpec((1, DF), lambda i: (0, 0)),
            pl.BlockSpec((1, DF), lambda i: (0, 0)),
            pl.BlockSpec((1, DF), lambda i: (0, 0)),
            pl.BlockSpec((1, DF), lambda i: (0, 0)),
        ],
        out_specs=[
            pl.BlockSpec((BN, AFL), lambda i: (i, 0)),
            pl.BlockSpec((BN, M, NBR), lambda i: (i, 0, 0)),
            pl.BlockSpec((1, AFL), lambda i: (0, 0)),
            pl.BlockSpec((1, AFL), lambda i: (0, 0)),
        ],
        out_shape=[
            jax.ShapeDtypeStruct((N, AFL), jnp.float32),
            jax.ShapeDtypeStruct((N, M, NBR), jnp.float32),
            jax.ShapeDtypeStruct((1, AFL), jnp.float32),
            jax.ShapeDtypeStruct((1, AFL), jnp.float32),
        ],
    )(g3, nbr_fea, s, q, g1.reshape(1, DF), b1.reshape(1, DF))

    BC = 1000
    out = pl.pallas_call(
        functools.partial(_bn2_body, n_total=float(N)),
        grid=(N // BC,),
        in_specs=[
            pl.BlockSpec((BC, AFL), lambda i: (i, 0)),
            pl.BlockSpec((BC, AFL), lambda i: (i, 0)),
            pl.BlockSpec((1, AFL), lambda i: (0, 0)),
            pl.BlockSpec((1, AFL), lambda i: (0, 0)),
            pl.BlockSpec((1, AFL), lambda i: (0, 0)),
            pl.BlockSpec((1, AFL), lambda i: (0, 0)),
        ],
        out_specs=pl.BlockSpec((BC, AFL), lambda i: (i, 0)),
        out_shape=jax.ShapeDtypeStruct((N, AFL), jnp.float32),
    )(atom_in_fea, ns, s2, q2, g2.reshape(1, AFL), b2.reshape(1, AFL))

    return (out, nn)
